# baseline (speedup 1.0000x reference)
import jax
import jax.numpy as jnp
from jax import lax
from jax.experimental import pallas as pl
from jax.experimental.pallas import tpu as pltpu, tpu_sc as plsc

_NT = 32
_SC_PARAMS = pltpu.CompilerParams(needs_layout_passes=False)


def _mesh():
    return plsc.VectorSubcoreMesh(core_axis_name="c", subcore_axis_name="s")


def _wid():
    return lax.axis_index("s") * 2 + lax.axis_index("c")


_Z16 = lambda: jnp.zeros((16,), jnp.float32)


def kernel(x, edge_index, edge_attr, W1, b1, W2, b2):
    src_e = edge_index[0]
    dst_e = edge_index[1]
    N, D_in = x.shape
    E = edge_attr.shape[0]
    D_h = W1.shape[1]
    D_out = W2.shape[1]
    f32 = jnp.float32

    EPT = E // _NT

    def deg_body(dst_hbm, w_hbm, out_hbm, dst_v, w_v, acc_v):
        wid = _wid()
        base = wid * EPT

        @plsc.parallel_loop(0, N, 16, unroll=8)
        def _zero(i):
            acc_v[pl.ds(i, 16)] = _Z16()

        pltpu.sync_copy(dst_hbm.at[pl.ds(base, EPT)], dst_v)
        pltpu.sync_copy(w_hbm.at[pl.ds(base, EPT)], w_v)

        @plsc.parallel_loop(0, EPT, 16, unroll=8)
        def _scat(j):
            d = dst_v[pl.ds(j, 16)]
            wv = w_v[pl.ds(j, 16)]
            plsc.addupdate_scatter(acc_v, [d], wv)

        pltpu.sync_copy(acc_v, out_hbm.at[pl.ds(wid * N, N)])

    deg_part = pl.kernel(
        deg_body,
        out_type=jax.ShapeDtypeStruct((_NT * N,), f32),
        mesh=_mesh(),
        scratch_types=[
            pltpu.VMEM((EPT,), jnp.int32),
            pltpu.VMEM((EPT,), f32),
            pltpu.VMEM((N,), f32),
        ],
        compiler_params=_SC_PARAMS,
    )(dst_e, edge_attr)

    W1e = W1[:, 0::2]
    W1o = W1[:, 1::2]

    def _pack_bf16(lo, hi):
        u0 = lax.bitcast_convert_type(lo, jnp.int32)
        u1 = lax.bitcast_convert_type(hi, jnp.int32)
        r0 = u0 + 0x7FFF + ((u0 >> 16) & 1)
        r1 = u1 + 0x7FFF + ((u1 >> 16) & 1)
        return (r1 & jnp.int32(-65536)) | ((r0 >> 16) & 0xFFFF)

    def tc1a(x_ref, w1e_ref, w1o_ref, h1t_ref, pk_ref):
        he = lax.dot_general(
            w1e_ref[...], x_ref[...], (((0,), (1,)), ((), ())),
            preferred_element_type=f32)
        ho = lax.dot_general(
            w1o_ref[...], x_ref[...], (((0,), (1,)), ((), ())),
            preferred_element_type=f32)
        h1t_ref[...] = jnp.concatenate([he, ho], axis=0)
        pk_ref[...] = _pack_bf16(he, ho)

    h1T, h1pk = pl.pallas_call(
        tc1a,
        out_shape=(jax.ShapeDtypeStruct((D_h, N), f32),
                   jax.ShapeDtypeStruct((D_h // 2, N), jnp.int32)),
    )(x, W1e, W1o)

    def tc1b(deg_ref, dinv_ref):
        deg = jnp.sum(deg_ref[...], axis=0) + 1.0
        dinv_ref[...] = lax.rsqrt(deg)

    dinv = pl.pallas_call(
        tc1b,
        out_shape=jax.ShapeDtypeStruct((N,), f32),
    )(deg_part.reshape(_NT, N))

    def norm_body(src_hbm, dst_hbm, w_hbm, dinv_hbm, nrm_hbm, pk_hbm,
                  src_v, dst_v, w_v, dinv_v, nrm_v, pk_v):
        wid = _wid()
        base = wid * EPT
        pltpu.sync_copy(dinv_hbm, dinv_v)
        pltpu.sync_copy(src_hbm.at[pl.ds(base, EPT)], src_v)
        pltpu.sync_copy(dst_hbm.at[pl.ds(base, EPT)], dst_v)
        pltpu.sync_copy(w_hbm.at[pl.ds(base, EPT)], w_v)

        @plsc.parallel_loop(0, EPT, 16, unroll=8)
        def _nrm(j):
            sl = pl.ds(j, 16)
            s = src_v[sl]
            d = dst_v[sl]
            wv = w_v[sl]
            nrm_v[sl] = plsc.load_gather(dinv_v, [s]) * wv * plsc.load_gather(dinv_v, [d])
            pk_v[sl] = (s << 16) | d

        pltpu.sync_copy(nrm_v, nrm_hbm.at[pl.ds(base, EPT)])
        pltpu.sync_copy(pk_v, pk_hbm.at[pl.ds(base, EPT)])

    norm, packed = pl.kernel(
        norm_body,
        out_type=(jax.ShapeDtypeStruct((E,), f32),
                  jax.ShapeDtypeStruct((E,), jnp.int32)),
        mesh=_mesh(),
        scratch_types=[
            pltpu.VMEM((EPT,), jnp.int32),
            pltpu.VMEM((EPT,), jnp.int32),
            pltpu.VMEM((EPT,), f32),
            pltpu.VMEM((N,), f32),
            pltpu.VMEM((EPT,), f32),
            pltpu.VMEM((EPT,), jnp.int32),
        ],
        compiler_params=_SC_PARAMS,
    )(src_e, dst_e, edge_attr, dinv)

    def make_agg(D, F, n_groups, n_reps, chunk):
        eps = E // n_reps
        nch = eps // chunk
        g_mask = n_groups - 1
        r_shift = n_groups.bit_length() - 1
        FN = F * N

        npairs = nch // 2
        assert nch % 2 == 0

        def body(pk_hbm, nrm_hbm, ht_hbm, out_hbm,
                 h_v, acc_v,
                 pk_v0, nrm_v0, pk_v1, nrm_v1,
                 sp0, sn0, sp1, sn1):
            wid = _wid()
            g = wid & g_mask
            r = wid >> r_shift
            ebase = r * eps
            E_tot = pk_hbm.shape[0]

            def issue(bufs, sems, ch):
                off = jnp.minimum(ebase + ch * chunk, E_tot - chunk)
                pltpu.async_copy(pk_hbm.at[pl.ds(off, chunk)], bufs[0], sems[0])
                pltpu.async_copy(nrm_hbm.at[pl.ds(off, chunk)], bufs[1], sems[1])

            def wait(bufs, sems):
                pltpu.make_async_copy(pk_hbm.at[pl.ds(0, chunk)], bufs[0], sems[0]).wait()
                pltpu.make_async_copy(nrm_hbm.at[pl.ds(0, chunk)], bufs[1], sems[1]).wait()

            def process(bufs):
                pk_v, nrm_v = bufs

                @plsc.parallel_loop(0, chunk, 16, unroll=4)
                def _agg(j):
                    sl = pl.ds(j, 16)
                    p = pk_v[sl]
                    s = p >> 16
                    d = p & 0xFFFF
                    nm = nrm_v[sl]
                    for f in range(F):
                        v = plsc.load_gather(h_v.at[pl.ds(f * N, N)], [s])
                        plsc.addupdate_scatter(acc_v.at[pl.ds(f * N, N)], [d], nm * v)

            b0 = (pk_v0, nrm_v0)
            b1 = (pk_v1, nrm_v1)
            s0 = (sp0, sn0)
            s1 = (sp1, sn1)

            issue(b0, s0, 0)
            pltpu.sync_copy(ht_hbm.at[pl.ds(g * FN, FN)], h_v)

            @plsc.parallel_loop(0, FN, 16, unroll=8)
            def _zero(i):
                acc_v[pl.ds(i, 16)] = _Z16()

            def pair_step(cp, _):
                ch0 = cp * 2
                issue(b1, s1, ch0 + 1)
                wait(b0, s0)
                process(b0)
                issue(b0, s0, ch0 + 2)
                wait(b1, s1)
                process(b1)
                return _

            lax.fori_loop(0, npairs, pair_step, None)
            wait(b0, s0)
            pltpu.sync_copy(acc_v, out_hbm.at[pl.ds(r * (D * N) + g * FN, FN)])

        return pl.kernel(
            body,
            out_type=jax.ShapeDtypeStruct((n_reps * D * N,), f32),
            mesh=_mesh(),
            scratch_types=[
                pltpu.VMEM((FN,), f32),
                pltpu.VMEM((FN,), f32),
                pltpu.VMEM((chunk,), jnp.int32),
                pltpu.VMEM((chunk,), f32),
                pltpu.VMEM((chunk,), jnp.int32),
                pltpu.VMEM((chunk,), f32),
                pltpu.SemaphoreType.DMA,
                pltpu.SemaphoreType.DMA,
                pltpu.SemaphoreType.DMA,
                pltpu.SemaphoreType.DMA,
            ],
            compiler_params=_SC_PARAMS,
        )

    def make_agg1(chunk):
        n_groups, n_reps = 16, 2
        eps = E // n_reps
        nch = eps // chunk
        assert nch % 2 == 0
        npairs = nch // 2
        N2 = 2 * N

        def body(pk_hbm, nrm_hbm, hpk_hbm, out_hbm,
                 h_v, acc_v, pk_v0, nrm_v0, pk_v1, nrm_v1, sp0, sn0, sp1, sn1):
            wid = _wid()
            g = wid & 15
            r = wid >> 4
            ebase = r * eps
            E_tot = pk_hbm.shape[0]

            def issue(bufs, sems, ch):
                off = jnp.minimum(ebase + ch * chunk, E_tot - chunk)
                pltpu.async_copy(pk_hbm.at[pl.ds(off, chunk)], bufs[0], sems[0])
                pltpu.async_copy(nrm_hbm.at[pl.ds(off, chunk)], bufs[1], sems[1])

            def wait(bufs, sems):
                pltpu.make_async_copy(pk_hbm.at[pl.ds(0, chunk)], bufs[0], sems[0]).wait()
                pltpu.make_async_copy(nrm_hbm.at[pl.ds(0, chunk)], bufs[1], sems[1]).wait()

            def process(bufs):
                pk_v, nrm_v = bufs

                @plsc.parallel_loop(0, chunk, 16, unroll=4)
                def _agg(j):
                    sl = pl.ds(j, 16)
                    p = pk_v[sl]
                    s = p >> 16
                    d = p & 0xFFFF
                    nm = nrm_v[sl]
                    for k in range(2):
                        pw = plsc.load_gather(h_v.at[pl.ds(k * N, N)], [s])
                        vlo = plsc.bitcast(pw << 16, f32)
                        vhi = plsc.bitcast(pw & jnp.int32(-65536), f32)
                        plsc.addupdate_scatter(acc_v.at[pl.ds(k * N, N)], [d], nm * vlo)
                        plsc.addupdate_scatter(acc_v.at[pl.ds((2 + k) * N, N)], [d], nm * vhi)

            b0 = (pk_v0, nrm_v0)
            b1 = (pk_v1, nrm_v1)
            s0 = (sp0, sn0)
            s1 = (sp1, sn1)
            issue(b0, s0, 0)
            pltpu.sync_copy(hpk_hbm.at[pl.ds(g * N2, N2)], h_v)

            @plsc.parallel_loop(0, 4 * N, 16, unroll=8)
            def _zero(i):
                acc_v[pl.ds(i, 16)] = _Z16()

            def pair_step(cp, _):
                ch0 = cp * 2
                issue(b1, s1, ch0 + 1)
                wait(b0, s0)
                process(b0)
                issue(b0, s0, ch0 + 2)
                wait(b1, s1)
                process(b1)
                return _

            lax.fori_loop(0, npairs, pair_step, None)
            wait(b0, s0)
            pltpu.sync_copy(acc_v.at[pl.ds(0, N2)],
                            out_hbm.at[pl.ds(r * (D_h * N) + (2 * g) * N, N2)])
            pltpu.sync_copy(acc_v.at[pl.ds(N2, N2)],
                            out_hbm.at[pl.ds(r * (D_h * N) + (32 + 2 * g) * N, N2)])

        return pl.kernel(
            body,
            out_type=jax.ShapeDtypeStruct((n_reps * D_h * N,), f32),
            mesh=_mesh(),
            scratch_types=[
                pltpu.VMEM((N2,), jnp.int32),
                pltpu.VMEM((4 * N,), f32),
                pltpu.VMEM((chunk,), jnp.int32),
                pltpu.VMEM((chunk,), f32),
                pltpu.VMEM((chunk,), jnp.int32),
                pltpu.VMEM((chunk,), f32),
                pltpu.SemaphoreType.DMA,
                pltpu.SemaphoreType.DMA,
                pltpu.SemaphoreType.DMA,
                pltpu.SemaphoreType.DMA,
            ],
            compiler_params=_SC_PARAMS,
        )

    p1 = make_agg1(16000)(packed, norm, h1pk.reshape(-1))

    perm = list(range(0, D_h, 2)) + list(range(1, D_h, 2))
    b1p = b1[jnp.array(perm)]
    W2r = W2[jnp.array(perm), :]

    lo_idx = [5 * g + k for g in range(8) for k in (0, 2, 4)]
    hi_idx = [min(5 * g + k, D_out - 1) for g in range(8) for k in (1, 3, 5)]
    W2lo = W2r[:, jnp.array(lo_idx)]
    W2hi = W2r[:, jnp.array(hi_idx)]

    def tc2(p_ref, h1t_ref, dinv_ref, w2_ref, w2lo_ref, w2hi_ref, b1_ref,
            h2t_ref, h2pk_ref):
        dinv = dinv_ref[...]
        o = p_ref[0] + p_ref[1] + (dinv * dinv) * h1t_ref[...]
        o = o + b1_ref[...][:, None]
        z = jnp.maximum(o, 0.0)
        h2t_ref[...] = lax.dot_general(
            w2_ref[...], z, (((0,), (0,)), ((), ())),
            preferred_element_type=f32)
        zlo = lax.dot_general(
            w2lo_ref[...], z, (((0,), (0,)), ((), ())),
            preferred_element_type=f32)
        zhi = lax.dot_general(
            w2hi_ref[...], z, (((0,), (0,)), ((), ())),
            preferred_element_type=f32)
        h2pk_ref[...] = _pack_bf16(zlo, zhi)

    h2T, h2pk = pl.pallas_call(
        tc2,
        out_shape=(jax.ShapeDtypeStruct((D_out, N), f32),
                   jax.ShapeDtypeStruct((24, N), jnp.int32)),
    )(p1.reshape(2, D_h, N), h1T, dinv, W2r, W2lo, W2hi, b1p)

    def make_agg2(chunk):
        n_groups, n_reps = 8, 4
        eps = E // n_reps
        nch = eps // chunk
        assert nch % 2 == 0
        npairs = nch // 2
        N3 = 3 * N
        N5 = 5 * N

        def body(pk_hbm, nrm_hbm, hpk_hbm, out_hbm,
                 h_v, acc_v, pk_v0, nrm_v0, pk_v1, nrm_v1, sp0, sn0, sp1, sn1):
            wid = _wid()
            g = wid & 7
            r = wid >> 3
            ebase = r * eps
            E_tot = pk_hbm.shape[0]

            def issue(bufs, sems, ch):
                off = jnp.minimum(ebase + ch * chunk, E_tot - chunk)
                pltpu.async_copy(pk_hbm.at[pl.ds(off, chunk)], bufs[0], sems[0])
                pltpu.async_copy(nrm_hbm.at[pl.ds(off, chunk)], bufs[1], sems[1])

            def wait(bufs, sems):
                pltpu.make_async_copy(pk_hbm.at[pl.ds(0, chunk)], bufs[0], sems[0]).wait()
                pltpu.make_async_copy(nrm_hbm.at[pl.ds(0, chunk)], bufs[1], sems[1]).wait()

            def process(bufs):
                pk_v, nrm_v = bufs

                @plsc.parallel_loop(0, chunk, 16, unroll=4)
                def _agg(j):
                    sl = pl.ds(j, 16)
                    p = pk_v[sl]
                    s = p >> 16
                    d = p & 0xFFFF
                    nm = nrm_v[sl]
                    for k in range(3):
                        pw = plsc.load_gather(h_v.at[pl.ds(k * N, N)], [s])
                        vlo = plsc.bitcast(pw << 16, f32)
                        plsc.addupdate_scatter(acc_v.at[pl.ds(2 * k * N, N)], [d], nm * vlo)
                        if k < 2:
                            vhi = plsc.bitcast(pw & jnp.int32(-65536), f32)
                            plsc.addupdate_scatter(acc_v.at[pl.ds((2 * k + 1) * N, N)], [d], nm * vhi)

            b0 = (pk_v0, nrm_v0)
            b1 = (pk_v1, nrm_v1)
            s0 = (sp0, sn0)
            s1 = (sp1, sn1)
            issue(b0, s0, 0)
            pltpu.sync_copy(hpk_hbm.at[pl.ds(g * N3, N3)], h_v)

            @plsc.parallel_loop(0, N5, 16, unroll=8)
            def _zero(i):
                acc_v[pl.ds(i, 16)] = _Z16()

            def pair_step(cp, _):
                ch0 = cp * 2
                issue(b1, s1, ch0 + 1)
                wait(b0, s0)
                process(b0)
                issue(b0, s0, ch0 + 2)
                wait(b1, s1)
                process(b1)
                return _

            lax.fori_loop(0, npairs, pair_step, None)
            wait(b0, s0)
            pltpu.sync_copy(acc_v, out_hbm.at[pl.ds(r * (D_out * N) + g * N5, N5)])

        return pl.kernel(
            body,
            out_type=jax.ShapeDtypeStruct((n_reps * D_out * N,), f32),
            mesh=_mesh(),
            scratch_types=[
                pltpu.VMEM((N3,), jnp.int32),
                pltpu.VMEM((N5,), f32),
                pltpu.VMEM((chunk,), jnp.int32),
                pltpu.VMEM((chunk,), f32),
                pltpu.VMEM((chunk,), jnp.int32),
                pltpu.VMEM((chunk,), f32),
                pltpu.SemaphoreType.DMA,
                pltpu.SemaphoreType.DMA,
                pltpu.SemaphoreType.DMA,
                pltpu.SemaphoreType.DMA,
            ],
            compiler_params=_SC_PARAMS,
        )

    p2 = make_agg2(8000)(packed, norm, h2pk.reshape(-1))

    def tc3(p_ref, h2t_ref, dinv_ref, b2_ref, out_ref):
        dinv = dinv_ref[...]
        o = p_ref[0] + p_ref[1] + p_ref[2] + p_ref[3]
        o = o + (dinv * dinv) * h2t_ref[...] + b2_ref[...][:, None]
        m = jnp.max(o, axis=0, keepdims=True)
        e = jnp.exp(o - m)
        sm = e / jnp.sum(e, axis=0, keepdims=True)
        out_ref[...] = jnp.transpose(sm, (1, 0))

    out = pl.pallas_call(
        tc3,
        out_shape=jax.ShapeDtypeStruct((N, D_out), f32),
    )(p2.reshape(4, D_out, N), h2T, dinv, b2)
    return out


# speedup vs baseline: 1.0039x; 1.0039x over previous
import jax
import jax.numpy as jnp
from jax import lax
from jax.experimental import pallas as pl
from jax.experimental.pallas import tpu as pltpu, tpu_sc as plsc

_NT = 32
_SC_PARAMS = pltpu.CompilerParams(needs_layout_passes=False)


def _mesh():
    return plsc.VectorSubcoreMesh(core_axis_name="c", subcore_axis_name="s")


def _wid():
    return lax.axis_index("s") * 2 + lax.axis_index("c")


_Z16 = lambda: jnp.zeros((16,), jnp.float32)


def kernel(x, edge_index, edge_attr, W1, b1, W2, b2):
    src_e = edge_index[0]
    dst_e = edge_index[1]
    N, D_in = x.shape
    E = edge_attr.shape[0]
    D_h = W1.shape[1]
    D_out = W2.shape[1]
    f32 = jnp.float32

    EPT = E // _NT

    def deg_body(dst_hbm, w_hbm, out_hbm, dst_v, w_v, acc_v):
        wid = _wid()
        base = wid * EPT

        @plsc.parallel_loop(0, N, 16, unroll=8)
        def _zero(i):
            acc_v[pl.ds(i, 16)] = _Z16()

        pltpu.sync_copy(dst_hbm.at[pl.ds(base, EPT)], dst_v)
        pltpu.sync_copy(w_hbm.at[pl.ds(base, EPT)], w_v)

        @plsc.parallel_loop(0, EPT, 16, unroll=8)
        def _scat(j):
            d = dst_v[pl.ds(j, 16)]
            wv = w_v[pl.ds(j, 16)]
            plsc.addupdate_scatter(acc_v, [d], wv)

        pltpu.sync_copy(acc_v, out_hbm.at[pl.ds(wid * N, N)])

    deg_part = pl.kernel(
        deg_body,
        out_type=jax.ShapeDtypeStruct((_NT * N,), f32),
        mesh=_mesh(),
        scratch_types=[
            pltpu.VMEM((EPT,), jnp.int32),
            pltpu.VMEM((EPT,), f32),
            pltpu.VMEM((N,), f32),
        ],
        compiler_params=_SC_PARAMS,
    )(dst_e, edge_attr)

    W1e = W1[:, 0::2]
    W1o = W1[:, 1::2]

    def _pack_bf16(lo, hi):
        u0 = lax.bitcast_convert_type(lo, jnp.int32)
        u1 = lax.bitcast_convert_type(hi, jnp.int32)
        r0 = u0 + 0x7FFF + ((u0 >> 16) & 1)
        r1 = u1 + 0x7FFF + ((u1 >> 16) & 1)
        return (r1 & jnp.int32(-65536)) | ((r0 >> 16) & 0xFFFF)

    def tc1a(x_ref, w1e_ref, w1o_ref, h1t_ref, pk_ref):
        he = lax.dot_general(
            w1e_ref[...], x_ref[...], (((0,), (1,)), ((), ())),
            preferred_element_type=f32)
        ho = lax.dot_general(
            w1o_ref[...], x_ref[...], (((0,), (1,)), ((), ())),
            preferred_element_type=f32)
        h1t_ref[...] = jnp.concatenate([he, ho], axis=0)
        pk_ref[...] = _pack_bf16(he, ho)

    h1T, h1pk = pl.pallas_call(
        tc1a,
        out_shape=(jax.ShapeDtypeStruct((D_h, N), f32),
                   jax.ShapeDtypeStruct((D_h // 2, N), jnp.int32)),
    )(x, W1e, W1o)

    def tc1b(deg_ref, dinv_ref):
        deg = jnp.sum(deg_ref[...], axis=0) + 1.0
        dinv_ref[...] = lax.rsqrt(deg)

    dinv = pl.pallas_call(
        tc1b,
        out_shape=jax.ShapeDtypeStruct((N,), f32),
    )(deg_part.reshape(_NT, N))

    def norm_body(src_hbm, dst_hbm, w_hbm, dinv_hbm, nrm_hbm, pk_hbm,
                  src_v, dst_v, w_v, dinv_v, nrm_v, pk_v):
        wid = _wid()
        base = wid * EPT
        pltpu.sync_copy(dinv_hbm, dinv_v)
        pltpu.sync_copy(src_hbm.at[pl.ds(base, EPT)], src_v)
        pltpu.sync_copy(dst_hbm.at[pl.ds(base, EPT)], dst_v)
        pltpu.sync_copy(w_hbm.at[pl.ds(base, EPT)], w_v)

        @plsc.parallel_loop(0, EPT, 16, unroll=8)
        def _nrm(j):
            sl = pl.ds(j, 16)
            s = src_v[sl]
            d = dst_v[sl]
            wv = w_v[sl]
            nrm_v[sl] = plsc.load_gather(dinv_v, [s]) * wv * plsc.load_gather(dinv_v, [d])
            pk_v[sl] = (s << 16) | d

        pltpu.sync_copy(nrm_v, nrm_hbm.at[pl.ds(base, EPT)])
        pltpu.sync_copy(pk_v, pk_hbm.at[pl.ds(base, EPT)])

    norm, packed = pl.kernel(
        norm_body,
        out_type=(jax.ShapeDtypeStruct((E,), f32),
                  jax.ShapeDtypeStruct((E,), jnp.int32)),
        mesh=_mesh(),
        scratch_types=[
            pltpu.VMEM((EPT,), jnp.int32),
            pltpu.VMEM((EPT,), jnp.int32),
            pltpu.VMEM((EPT,), f32),
            pltpu.VMEM((N,), f32),
            pltpu.VMEM((EPT,), f32),
            pltpu.VMEM((EPT,), jnp.int32),
        ],
        compiler_params=_SC_PARAMS,
    )(src_e, dst_e, edge_attr, dinv)

    def make_agg(D, F, n_groups, n_reps, chunk):
        eps = E // n_reps
        nch = eps // chunk
        g_mask = n_groups - 1
        r_shift = n_groups.bit_length() - 1
        FN = F * N

        npairs = nch // 2
        assert nch % 2 == 0

        def body(pk_hbm, nrm_hbm, ht_hbm, out_hbm,
                 h_v, acc_v,
                 pk_v0, nrm_v0, pk_v1, nrm_v1,
                 sp0, sn0, sp1, sn1):
            wid = _wid()
            g = wid & g_mask
            r = wid >> r_shift
            ebase = r * eps
            E_tot = pk_hbm.shape[0]

            def issue(bufs, sems, ch):
                off = jnp.minimum(ebase + ch * chunk, E_tot - chunk)
                pltpu.async_copy(pk_hbm.at[pl.ds(off, chunk)], bufs[0], sems[0])
                pltpu.async_copy(nrm_hbm.at[pl.ds(off, chunk)], bufs[1], sems[1])

            def wait(bufs, sems):
                pltpu.make_async_copy(pk_hbm.at[pl.ds(0, chunk)], bufs[0], sems[0]).wait()
                pltpu.make_async_copy(nrm_hbm.at[pl.ds(0, chunk)], bufs[1], sems[1]).wait()

            def process(bufs):
                pk_v, nrm_v = bufs

                @plsc.parallel_loop(0, chunk, 16, unroll=4)
                def _agg(j):
                    sl = pl.ds(j, 16)
                    p = pk_v[sl]
                    s = p >> 16
                    d = p & 0xFFFF
                    nm = nrm_v[sl]
                    for f in range(F):
                        v = plsc.load_gather(h_v.at[pl.ds(f * N, N)], [s])
                        plsc.addupdate_scatter(acc_v.at[pl.ds(f * N, N)], [d], nm * v)

            b0 = (pk_v0, nrm_v0)
            b1 = (pk_v1, nrm_v1)
            s0 = (sp0, sn0)
            s1 = (sp1, sn1)

            issue(b0, s0, 0)
            pltpu.sync_copy(ht_hbm.at[pl.ds(g * FN, FN)], h_v)

            @plsc.parallel_loop(0, FN, 16, unroll=8)
            def _zero(i):
                acc_v[pl.ds(i, 16)] = _Z16()

            def pair_step(cp, _):
                ch0 = cp * 2
                issue(b1, s1, ch0 + 1)
                wait(b0, s0)
                process(b0)
                issue(b0, s0, ch0 + 2)
                wait(b1, s1)
                process(b1)
                return _

            lax.fori_loop(0, npairs, pair_step, None)
            wait(b0, s0)
            pltpu.sync_copy(acc_v, out_hbm.at[pl.ds(r * (D * N) + g * FN, FN)])

        return pl.kernel(
            body,
            out_type=jax.ShapeDtypeStruct((n_reps * D * N,), f32),
            mesh=_mesh(),
            scratch_types=[
                pltpu.VMEM((FN,), f32),
                pltpu.VMEM((FN,), f32),
                pltpu.VMEM((chunk,), jnp.int32),
                pltpu.VMEM((chunk,), f32),
                pltpu.VMEM((chunk,), jnp.int32),
                pltpu.VMEM((chunk,), f32),
                pltpu.SemaphoreType.DMA,
                pltpu.SemaphoreType.DMA,
                pltpu.SemaphoreType.DMA,
                pltpu.SemaphoreType.DMA,
            ],
            compiler_params=_SC_PARAMS,
        )

    def make_agg1(chunk):
        n_groups, n_reps = 16, 2
        eps = E // n_reps
        nch = eps // chunk
        assert nch % 2 == 0
        npairs = nch // 2
        N2 = 2 * N

        def body(pk_hbm, nrm_hbm, hpk_hbm, out_hbm,
                 h_v, acc_v, pk_v0, nrm_v0, pk_v1, nrm_v1, sp0, sn0, sp1, sn1):
            wid = _wid()
            g = wid & 15
            r = wid >> 4
            ebase = r * eps
            E_tot = pk_hbm.shape[0]

            def issue(bufs, sems, ch):
                off = jnp.minimum(ebase + ch * chunk, E_tot - chunk)
                pltpu.async_copy(pk_hbm.at[pl.ds(off, chunk)], bufs[0], sems[0])
                pltpu.async_copy(nrm_hbm.at[pl.ds(off, chunk)], bufs[1], sems[1])

            def wait(bufs, sems):
                pltpu.make_async_copy(pk_hbm.at[pl.ds(0, chunk)], bufs[0], sems[0]).wait()
                pltpu.make_async_copy(nrm_hbm.at[pl.ds(0, chunk)], bufs[1], sems[1]).wait()

            def process(bufs):
                pk_v, nrm_v = bufs

                @plsc.parallel_loop(0, chunk, 16, unroll=4)
                def _agg(j):
                    sl = pl.ds(j, 16)
                    p = pk_v[sl]
                    s = p >> 16
                    d = p & 0xFFFF
                    nm = nrm_v[sl]
                    for k in range(2):
                        pw = plsc.load_gather(h_v.at[pl.ds(k * N, N)], [s])
                        vlo = plsc.bitcast(pw << 16, f32)
                        vhi = plsc.bitcast(pw & jnp.int32(-65536), f32)
                        plsc.addupdate_scatter(acc_v.at[pl.ds(k * N, N)], [d], nm * vlo)
                        plsc.addupdate_scatter(acc_v.at[pl.ds((2 + k) * N, N)], [d], nm * vhi)

            b0 = (pk_v0, nrm_v0)
            b1 = (pk_v1, nrm_v1)
            s0 = (sp0, sn0)
            s1 = (sp1, sn1)
            issue(b0, s0, 0)
            pltpu.sync_copy(hpk_hbm.at[pl.ds(g * N2, N2)], h_v)

            @plsc.parallel_loop(0, 4 * N, 16, unroll=8)
            def _zero(i):
                acc_v[pl.ds(i, 16)] = _Z16()

            def pair_step(cp, _):
                ch0 = cp * 2
                issue(b1, s1, ch0 + 1)
                wait(b0, s0)
                process(b0)
                issue(b0, s0, ch0 + 2)
                wait(b1, s1)
                process(b1)
                return _

            lax.fori_loop(0, npairs, pair_step, None)
            wait(b0, s0)
            pltpu.sync_copy(acc_v.at[pl.ds(0, N2)],
                            out_hbm.at[pl.ds(r * (D_h * N) + (2 * g) * N, N2)])
            pltpu.sync_copy(acc_v.at[pl.ds(N2, N2)],
                            out_hbm.at[pl.ds(r * (D_h * N) + (32 + 2 * g) * N, N2)])

        return pl.kernel(
            body,
            out_type=jax.ShapeDtypeStruct((n_reps * D_h * N,), f32),
            mesh=_mesh(),
            scratch_types=[
                pltpu.VMEM((N2,), jnp.int32),
                pltpu.VMEM((4 * N,), f32),
                pltpu.VMEM((chunk,), jnp.int32),
                pltpu.VMEM((chunk,), f32),
                pltpu.VMEM((chunk,), jnp.int32),
                pltpu.VMEM((chunk,), f32),
                pltpu.SemaphoreType.DMA,
                pltpu.SemaphoreType.DMA,
                pltpu.SemaphoreType.DMA,
                pltpu.SemaphoreType.DMA,
            ],
            compiler_params=_SC_PARAMS,
        )

    p1 = make_agg1(8000)(packed, norm, h1pk.reshape(-1))

    perm = list(range(0, D_h, 2)) + list(range(1, D_h, 2))
    b1p = b1[jnp.array(perm)]
    W2r = W2[jnp.array(perm), :]

    lo_idx = [5 * g + k for g in range(8) for k in (0, 2, 4)]
    hi_idx = [min(5 * g + k, D_out - 1) for g in range(8) for k in (1, 3, 5)]
    W2lo = W2r[:, jnp.array(lo_idx)]
    W2hi = W2r[:, jnp.array(hi_idx)]

    def tc2(p_ref, h1t_ref, dinv_ref, w2_ref, w2lo_ref, w2hi_ref, b1_ref,
            h2t_ref, h2pk_ref):
        dinv = dinv_ref[...]
        o = p_ref[0] + p_ref[1] + (dinv * dinv) * h1t_ref[...]
        o = o + b1_ref[...][:, None]
        z = jnp.maximum(o, 0.0)
        h2t_ref[...] = lax.dot_general(
            w2_ref[...], z, (((0,), (0,)), ((), ())),
            preferred_element_type=f32)
        zlo = lax.dot_general(
            w2lo_ref[...], z, (((0,), (0,)), ((), ())),
            preferred_element_type=f32)
        zhi = lax.dot_general(
            w2hi_ref[...], z, (((0,), (0,)), ((), ())),
            preferred_element_type=f32)
        h2pk_ref[...] = _pack_bf16(zlo, zhi)

    h2T, h2pk = pl.pallas_call(
        tc2,
        out_shape=(jax.ShapeDtypeStruct((D_out, N), f32),
                   jax.ShapeDtypeStruct((24, N), jnp.int32)),
    )(p1.reshape(2, D_h, N), h1T, dinv, W2r, W2lo, W2hi, b1p)

    def make_agg2(chunk):
        n_groups, n_reps = 8, 4
        eps = E // n_reps
        nch = eps // chunk
        assert nch % 2 == 0
        npairs = nch // 2
        N3 = 3 * N
        N5 = 5 * N

        def body(pk_hbm, nrm_hbm, hpk_hbm, out_hbm,
                 h_v, acc_v, pk_v0, nrm_v0, pk_v1, nrm_v1, sp0, sn0, sp1, sn1):
            wid = _wid()
            g = wid & 7
            r = wid >> 3
            ebase = r * eps
            E_tot = pk_hbm.shape[0]

            def issue(bufs, sems, ch):
                off = jnp.minimum(ebase + ch * chunk, E_tot - chunk)
                pltpu.async_copy(pk_hbm.at[pl.ds(off, chunk)], bufs[0], sems[0])
                pltpu.async_copy(nrm_hbm.at[pl.ds(off, chunk)], bufs[1], sems[1])

            def wait(bufs, sems):
                pltpu.make_async_copy(pk_hbm.at[pl.ds(0, chunk)], bufs[0], sems[0]).wait()
                pltpu.make_async_copy(nrm_hbm.at[pl.ds(0, chunk)], bufs[1], sems[1]).wait()

            def process(bufs):
                pk_v, nrm_v = bufs

                @plsc.parallel_loop(0, chunk, 16, unroll=4)
                def _agg(j):
                    sl = pl.ds(j, 16)
                    p = pk_v[sl]
                    s = p >> 16
                    d = p & 0xFFFF
                    nm = nrm_v[sl]
                    for k in range(3):
                        pw = plsc.load_gather(h_v.at[pl.ds(k * N, N)], [s])
                        vlo = plsc.bitcast(pw << 16, f32)
                        plsc.addupdate_scatter(acc_v.at[pl.ds(2 * k * N, N)], [d], nm * vlo)
                        if k < 2:
                            vhi = plsc.bitcast(pw & jnp.int32(-65536), f32)
                            plsc.addupdate_scatter(acc_v.at[pl.ds((2 * k + 1) * N, N)], [d], nm * vhi)

            b0 = (pk_v0, nrm_v0)
            b1 = (pk_v1, nrm_v1)
            s0 = (sp0, sn0)
            s1 = (sp1, sn1)
            issue(b0, s0, 0)
            pltpu.sync_copy(hpk_hbm.at[pl.ds(g * N3, N3)], h_v)

            @plsc.parallel_loop(0, N5, 16, unroll=8)
            def _zero(i):
                acc_v[pl.ds(i, 16)] = _Z16()

            def pair_step(cp, _):
                ch0 = cp * 2
                issue(b1, s1, ch0 + 1)
                wait(b0, s0)
                process(b0)
                issue(b0, s0, ch0 + 2)
                wait(b1, s1)
                process(b1)
                return _

            lax.fori_loop(0, npairs, pair_step, None)
            wait(b0, s0)
            pltpu.sync_copy(acc_v, out_hbm.at[pl.ds(r * (D_out * N) + g * N5, N5)])

        return pl.kernel(
            body,
            out_type=jax.ShapeDtypeStruct((n_reps * D_out * N,), f32),
            mesh=_mesh(),
            scratch_types=[
                pltpu.VMEM((N3,), jnp.int32),
                pltpu.VMEM((N5,), f32),
                pltpu.VMEM((chunk,), jnp.int32),
                pltpu.VMEM((chunk,), f32),
                pltpu.VMEM((chunk,), jnp.int32),
                pltpu.VMEM((chunk,), f32),
                pltpu.SemaphoreType.DMA,
                pltpu.SemaphoreType.DMA,
                pltpu.SemaphoreType.DMA,
                pltpu.SemaphoreType.DMA,
            ],
            compiler_params=_SC_PARAMS,
        )

    p2 = make_agg2(8000)(packed, norm, h2pk.reshape(-1))

    def tc3(p_ref, h2t_ref, dinv_ref, b2_ref, out_ref):
        dinv = dinv_ref[...]
        o = p_ref[0] + p_ref[1] + p_ref[2] + p_ref[3]
        o = o + (dinv * dinv) * h2t_ref[...] + b2_ref[...][:, None]
        m = jnp.max(o, axis=0, keepdims=True)
        e = jnp.exp(o - m)
        sm = e / jnp.sum(e, axis=0, keepdims=True)
        out_ref[...] = jnp.transpose(sm, (1, 0))

    out = pl.pallas_call(
        tc3,
        out_shape=jax.ShapeDtypeStruct((N, D_out), f32),
    )(p2.reshape(4, D_out, N), h2T, dinv, b2)
    return out
